# baseline (speedup 1.0000x reference)
import functools

import jax
import jax.numpy as jnp
from jax import lax
from jax.experimental import pallas as pl
from jax.experimental.pallas import tpu as pltpu
from jax.experimental.pallas import tpu_sc as plsc

R = 16
N = 100000
B = 16384
REG_PARA = 0.01
NC, NS, L = 2, 16, 16
NW = NC * NS
PER_W = B // NW
C = 64
NCH = PER_W // C


def _sc_body(idx_in, w0, p0, w1, p1, w2, p2, out, partials,
             ixall, ixb0, ixb1, ixb2,
             r0w, r0p, r1w, r1p, r2w, r2p,
             obuf, regbuf, sem):
    wid = lax.axis_index("c") * NS + lax.axis_index("s")
    base = wid * PER_W

    zeros = jnp.zeros((L,), jnp.float32)
    lanes = lax.iota(jnp.int32, L)
    col0 = lanes * 0

    pltpu.sync_copy(idx_in.at[pl.ds(base, PER_W)], ixall)

    def decol(g, carry):
        rows = g * L + lanes
        ixb0[pl.ds(g * L, L)] = plsc.load_gather(ixall, [rows, col0])
        ixb1[pl.ds(g * L, L)] = plsc.load_gather(ixall, [rows, col0 + 1])
        ixb2[pl.ds(g * L, L)] = plsc.load_gather(ixall, [rows, col0 + 2])
        return carry
    lax.fori_loop(0, PER_W // L, decol, 0)

    def subchunk(j, carry):
        s0, s1, s2 = carry
        o = j * C
        i0 = ixb0.at[pl.ds(o, C)]
        i1 = ixb1.at[pl.ds(o, C)]
        i2 = ixb2.at[pl.ds(o, C)]

        copies = [
            pltpu.async_copy(w0.at[i0], r0w, sem),
            pltpu.async_copy(p0.at[i0], r0p, sem),
            pltpu.async_copy(w1.at[i1], r1w, sem),
            pltpu.async_copy(p1.at[i1], r1p, sem),
            pltpu.async_copy(w2.at[i2], r2w, sem),
            pltpu.async_copy(p2.at[i2], r2p, sem),
        ]
        for cp in copies:
            cp.wait()

        def group(g, c2):
            s0, s1, s2 = c2
            outv = zeros
            for i in range(L):
                b = g * L + i
                v0 = r0w[b, :] * r0p[b, :]
                v2 = r2w[b, :] * r2p[b, :]
                t = zeros
                msum = zeros
                for r in range(R):
                    m = r1w[b, r, :] * r1p[b, r, :]
                    msum = msum + m
                    t = t + v0[r] * m
                sval = jnp.sum(t * v2)
                outv = jnp.where(lanes == i, sval, outv)
                s0 = s0 + v0
                s1 = s1 + msum
                s2 = s2 + v2
            obuf[pl.ds(g * L, L)] = outv
            return (s0, s1, s2)

        s0, s1, s2 = lax.fori_loop(0, C // L, group, (s0, s1, s2))
        pltpu.sync_copy(obuf, out.at[pl.ds(base + o, C)])
        return (s0, s1, s2)

    s0, s1, s2 = lax.fori_loop(0, NCH, subchunk, (zeros, zeros, zeros))

    regbuf[0, :] = s0
    regbuf[1, :] = s1
    regbuf[2, :] = s2
    pltpu.sync_copy(regbuf, partials.at[wid])


@jax.jit
def _tt_lookup(indices, w0, p0, w1, p1, w2, p2):
    mesh = plsc.VectorSubcoreMesh(core_axis_name="c", subcore_axis_name="s")
    f = pl.kernel(
        _sc_body,
        out_type=[
            jax.ShapeDtypeStruct((B,), jnp.float32),
            jax.ShapeDtypeStruct((NW, 3, L), jnp.float32),
        ],
        mesh=mesh,
        compiler_params=pltpu.CompilerParams(
            needs_layout_passes=False, use_tc_tiling_on_sc=False),
        scratch_types=[
            pltpu.VMEM((PER_W, 3), jnp.int32),
            pltpu.VMEM((PER_W,), jnp.int32),
            pltpu.VMEM((PER_W,), jnp.int32),
            pltpu.VMEM((PER_W,), jnp.int32),
            pltpu.VMEM((C, R), jnp.float32),
            pltpu.VMEM((C, R), jnp.float32),
            pltpu.VMEM((C, R, R), jnp.float32),
            pltpu.VMEM((C, R, R), jnp.float32),
            pltpu.VMEM((C, R), jnp.float32),
            pltpu.VMEM((C, R), jnp.float32),
            pltpu.VMEM((C,), jnp.float32),
            pltpu.VMEM((3, L), jnp.float32),
            pltpu.SemaphoreType.DMA,
        ],
    )
    return f(indices, w0, p0, w1, p1, w2, p2)


def kernel(indices, W0, W1, W2, P0, P1, P2):
    w0 = W0[0]
    p0 = P0[0]
    w1 = jnp.transpose(W1, (1, 0, 2))
    p1 = jnp.transpose(P1, (1, 0, 2))
    w2 = jnp.transpose(W2[:, :, 0], (1, 0))
    p2 = jnp.transpose(P2[:, :, 0], (1, 0))
    preds, partials = _tt_lookup(indices, w0, p0, w1, p1, w2, p2)
    s = jnp.sum(partials, axis=(0, 2))
    reg = REG_PARA * (jnp.abs(s[0]) / (B * R)
                      + jnp.abs(s[1]) / (B * R * R)
                      + jnp.abs(s[2]) / (B * R))
    return preds, reg


# speedup vs baseline: 3.2123x; 3.2123x over previous
import functools

import jax
import jax.numpy as jnp
from jax import lax
from jax.experimental import pallas as pl
from jax.experimental.pallas import tpu as pltpu
from jax.experimental.pallas import tpu_sc as plsc

R = 16
N = 100000
B = 16384
REG_PARA = 0.01
NC, NS, L = 2, 16, 16
NW = NC * NS
PER_W = B // NW
C = 64
NCH = PER_W // C


def _sc_body(ix0, ix1, ix2, w1v, p1v, q02, out, partials,
             ixb0, ixb1, ixb2,
             r1w, r1p, r02a, r02b,
             obuf, regbuf, sem):
    wid = lax.axis_index("c") * NS + lax.axis_index("s")
    base = wid * PER_W

    zeros = jnp.zeros((L,), jnp.float32)
    lanes = lax.iota(jnp.int32, L)

    pltpu.sync_copy(ix0.at[pl.ds(base, PER_W)], ixb0)
    pltpu.sync_copy(ix1.at[pl.ds(base, PER_W)], ixb1)
    pltpu.sync_copy(ix2.at[pl.ds(base, PER_W)], ixb2)

    def subchunk(j, carry):
        s0, s1, s2 = carry
        o = j * C

        copies = [
            pltpu.async_copy(w1v.at[ixb1.at[pl.ds(o, C)]], r1w, sem),
            pltpu.async_copy(p1v.at[ixb1.at[pl.ds(o, C)]], r1p, sem),
            pltpu.async_copy(q02.at[ixb0.at[pl.ds(o, C)]], r02a, sem),
            pltpu.async_copy(q02.at[ixb2.at[pl.ds(o, C)]], r02b, sem),
        ]
        for cp in copies:
            cp.wait()

        def group(g, c2):
            s0, s1, s2 = c2
            outv = zeros
            for i in range(L):
                b = g * L + i
                v0 = r02a[b, pl.ds(0, L)] * r02a[b, pl.ds(L, L)]
                v2 = r02b[b, pl.ds(2 * L, L)] * r02b[b, pl.ds(3 * L, L)]
                t = zeros
                msum = zeros
                for r in range(R):
                    m = (r1w[b, pl.ds(r * L, L)]
                         * r1p[b, pl.ds(r * L, L)])
                    msum = msum + m
                    t = t + v0[r] * m
                sval = jnp.sum(t * v2)
                outv = jnp.where(lanes == i, sval, outv)
                s0 = s0 + v0
                s1 = s1 + msum
                s2 = s2 + v2
            obuf[pl.ds(g * L, L)] = outv
            return (s0, s1, s2)

        s0, s1, s2 = lax.fori_loop(0, C // L, group, (s0, s1, s2))
        pltpu.sync_copy(obuf, out.at[pl.ds(base + o, C)])
        return (s0, s1, s2)

    s0, s1, s2 = lax.fori_loop(0, NCH, subchunk, (zeros, zeros, zeros))

    regbuf[0, :] = s0
    regbuf[1, :] = s1
    regbuf[2, :] = s2
    pltpu.sync_copy(regbuf, partials.at[wid])


@jax.jit
def _tt_lookup(ix0, ix1, ix2, w1v, p1v, q02):
    mesh = plsc.VectorSubcoreMesh(core_axis_name="c", subcore_axis_name="s")
    f = pl.kernel(
        _sc_body,
        out_type=[
            jax.ShapeDtypeStruct((B,), jnp.float32),
            jax.ShapeDtypeStruct((NW, 3, L), jnp.float32),
        ],
        mesh=mesh,
        compiler_params=pltpu.CompilerParams(
            needs_layout_passes=False, use_tc_tiling_on_sc=True),
        scratch_types=[
            pltpu.VMEM((PER_W,), jnp.int32),
            pltpu.VMEM((PER_W,), jnp.int32),
            pltpu.VMEM((PER_W,), jnp.int32),
            pltpu.VMEM((C, R * R), jnp.float32),
            pltpu.VMEM((C, R * R), jnp.float32),
            pltpu.VMEM((C, 8 * L), jnp.float32),
            pltpu.VMEM((C, 8 * L), jnp.float32),
            pltpu.VMEM((C,), jnp.float32),
            pltpu.VMEM((3, L), jnp.float32),
            pltpu.SemaphoreType.DMA,
        ],
    )
    return f(ix0, ix1, ix2, w1v, p1v, q02)


def kernel(indices, W0, W1, W2, P0, P1, P2):
    ix0 = indices[:, 0]
    ix1 = indices[:, 1]
    ix2 = indices[:, 2]
    w1v = jnp.transpose(W1, (1, 0, 2)).reshape(N, R * R)
    p1v = jnp.transpose(P1, (1, 0, 2)).reshape(N, R * R)
    q02 = jnp.concatenate(
        [W0[0], P0[0],
         jnp.transpose(W2[:, :, 0], (1, 0)),
         jnp.transpose(P2[:, :, 0], (1, 0)),
         jnp.zeros((N, 4 * R), jnp.float32)], axis=1)
    preds, partials = _tt_lookup(ix0, ix1, ix2, w1v, p1v, q02)
    s = jnp.sum(partials, axis=(0, 2))
    reg = REG_PARA * (jnp.abs(s[0]) / (B * R)
                      + jnp.abs(s[1]) / (B * R * R)
                      + jnp.abs(s[2]) / (B * R))
    return preds, reg


# speedup vs baseline: 4.5857x; 1.4276x over previous
import functools

import jax
import jax.numpy as jnp
from jax import lax
from jax.experimental import pallas as pl
from jax.experimental.pallas import tpu as pltpu
from jax.experimental.pallas import tpu_sc as plsc

R = 16
N = 100000
B = 16384
REG_PARA = 0.01
NC, NS, L = 2, 16, 16
NW = NC * NS
PER_W = B // NW
C = 64
NCH = PER_W // C


def _sc_body(ix0, ix1, ix2, q1, q02, out, partials,
             ixb0, ixb1, ixb2,
             r1w, r02a, r02b,
             obuf, regbuf, sem):
    wid = lax.axis_index("c") * NS + lax.axis_index("s")
    base = wid * PER_W

    zeros = jnp.zeros((L,), jnp.float32)
    lanes = lax.iota(jnp.int32, L)

    pltpu.sync_copy(ix0.at[pl.ds(base, PER_W)], ixb0)
    pltpu.sync_copy(ix1.at[pl.ds(base, PER_W)], ixb1)
    pltpu.sync_copy(ix2.at[pl.ds(base, PER_W)], ixb2)

    def subchunk(j, carry):
        s0, s1, s2 = carry
        o = j * C

        copies = [
            pltpu.async_copy(q1.at[ixb1.at[pl.ds(o, C)]], r1w, sem),
            pltpu.async_copy(q02.at[ixb0.at[pl.ds(o, C)]], r02a, sem),
            pltpu.async_copy(q02.at[ixb2.at[pl.ds(o, C)]], r02b, sem),
        ]
        for cp in copies:
            cp.wait()

        def group(g, c2):
            s0, s1, s2 = c2
            outv = zeros
            for i in range(L):
                b = g * L + i
                v0 = r02a[b, pl.ds(0, L)]
                v2 = r02b[b, pl.ds(L, L)]
                t = zeros
                msum = zeros
                for r in range(R):
                    m = r1w[b, pl.ds(r * L, L)]
                    msum = msum + m
                    t = t + v0[r] * m
                sval = jnp.sum(t * v2)
                outv = jnp.where(lanes == i, sval, outv)
                s0 = s0 + v0
                s1 = s1 + msum
                s2 = s2 + v2
            obuf[pl.ds(g * L, L)] = outv
            return (s0, s1, s2)

        s0, s1, s2 = lax.fori_loop(0, C // L, group, (s0, s1, s2))
        pltpu.sync_copy(obuf, out.at[pl.ds(base + o, C)])
        return (s0, s1, s2)

    s0, s1, s2 = lax.fori_loop(0, NCH, subchunk, (zeros, zeros, zeros))

    regbuf[0, :] = s0
    regbuf[1, :] = s1
    regbuf[2, :] = s2
    pltpu.sync_copy(regbuf, partials.at[wid])


@jax.jit
def _tt_lookup(ix0, ix1, ix2, q1, q02):
    mesh = plsc.VectorSubcoreMesh(core_axis_name="c", subcore_axis_name="s")
    f = pl.kernel(
        _sc_body,
        out_type=[
            jax.ShapeDtypeStruct((B,), jnp.float32),
            jax.ShapeDtypeStruct((NW, 3, L), jnp.float32),
        ],
        mesh=mesh,
        compiler_params=pltpu.CompilerParams(
            needs_layout_passes=False, use_tc_tiling_on_sc=True),
        scratch_types=[
            pltpu.VMEM((PER_W,), jnp.int32),
            pltpu.VMEM((PER_W,), jnp.int32),
            pltpu.VMEM((PER_W,), jnp.int32),
            pltpu.VMEM((C, R * R), jnp.float32),
            pltpu.VMEM((C, 8 * L), jnp.float32),
            pltpu.VMEM((C, 8 * L), jnp.float32),
            pltpu.VMEM((C,), jnp.float32),
            pltpu.VMEM((3, L), jnp.float32),
            pltpu.SemaphoreType.DMA,
        ],
    )
    return f(ix0, ix1, ix2, q1, q02)


def kernel(indices, W0, W1, W2, P0, P1, P2):
    ix0 = indices[:, 0]
    ix1 = indices[:, 1]
    ix2 = indices[:, 2]
    q1 = jnp.transpose(W1 * P1, (1, 0, 2)).reshape(N, R * R)
    q02 = jnp.concatenate(
        [W0[0] * P0[0],
         jnp.transpose(W2[:, :, 0] * P2[:, :, 0], (1, 0)),
         jnp.zeros((N, 6 * R), jnp.float32)], axis=1)
    preds, partials = _tt_lookup(ix0, ix1, ix2, q1, q02)
    s = jnp.sum(partials, axis=(0, 2))
    reg = REG_PARA * (jnp.abs(s[0]) / (B * R)
                      + jnp.abs(s[1]) / (B * R * R)
                      + jnp.abs(s[2]) / (B * R))
    return preds, reg


# speedup vs baseline: 4.6969x; 1.0242x over previous
import functools

import jax
import jax.numpy as jnp
from jax import lax
from jax.experimental import pallas as pl
from jax.experimental.pallas import tpu as pltpu
from jax.experimental.pallas import tpu_sc as plsc

R = 16
N = 100000
B = 16384
REG_PARA = 0.01
NC, NS, L = 2, 16, 16
NW = NC * NS
PER_W = B // NW
C = 64
NCH = PER_W // C


def _sc_body(ix0, ix1, ix2, q1, q02, out, partials,
             ixb0, ixb1, ixb2,
             r1wA, r02aA, r02bA, r1wB, r02aB, r02bB,
             obuf, regbuf, semA, semB):
    wid = lax.axis_index("c") * NS + lax.axis_index("s")
    base = wid * PER_W

    zeros = jnp.zeros((L,), jnp.float32)
    lanes = lax.iota(jnp.int32, L)

    pltpu.sync_copy(ix0.at[pl.ds(base, PER_W)], ixb0)
    pltpu.sync_copy(ix1.at[pl.ds(base, PER_W)], ixb1)
    pltpu.sync_copy(ix2.at[pl.ds(base, PER_W)], ixb2)

    def mk(jj, r1_, a_, b_, sem_):
        o = jj * C
        return [
            pltpu.make_async_copy(q1.at[ixb1.at[pl.ds(o, C)]], r1_, sem_),
            pltpu.make_async_copy(q02.at[ixb0.at[pl.ds(o, C)]], a_, sem_),
            pltpu.make_async_copy(q02.at[ixb2.at[pl.ds(o, C)]], b_, sem_),
        ]

    def compute(jj, r1_, a_, b_, c2):
        s0, s1, s2 = c2

        def group(g, c3):
            s0, s1, s2 = c3
            outv = zeros
            for i in range(L):
                b = g * L + i
                v0 = a_[b, pl.ds(0, L)]
                v2 = b_[b, pl.ds(L, L)]
                t = zeros
                msum = zeros
                for r in range(R):
                    m = r1_[b, pl.ds(r * L, L)]
                    msum = msum + m
                    t = t + v0[r] * m
                sval = jnp.sum(t * v2)
                outv = jnp.where(lanes == i, sval, outv)
                s0 = s0 + v0
                s1 = s1 + msum
                s2 = s2 + v2
            obuf[pl.ds(g * L, L)] = outv
            return (s0, s1, s2)

        c2 = lax.fori_loop(0, C // L, group, (s0, s1, s2))
        pltpu.sync_copy(obuf, out.at[pl.ds(base + jj * C, C)])
        return c2

    for cp in mk(0, r1wA, r02aA, r02bA, semA):
        cp.start()

    def pair(t, carry):
        ja = 2 * t
        for cp in mk(ja + 1, r1wB, r02aB, r02bB, semB):
            cp.start()
        for cp in mk(ja, r1wA, r02aA, r02bA, semA):
            cp.wait()
        carry = compute(ja, r1wA, r02aA, r02bA, carry)

        @pl.when(t < NCH // 2 - 1)
        def _():
            for cp in mk(ja + 2, r1wA, r02aA, r02bA, semA):
                cp.start()

        for cp in mk(ja + 1, r1wB, r02aB, r02bB, semB):
            cp.wait()
        carry = compute(ja + 1, r1wB, r02aB, r02bB, carry)
        return carry

    s0, s1, s2 = lax.fori_loop(0, NCH // 2, pair, (zeros, zeros, zeros))

    regbuf[0, :] = s0
    regbuf[1, :] = s1
    regbuf[2, :] = s2
    pltpu.sync_copy(regbuf, partials.at[wid])


@jax.jit
def _tt_lookup(ix0, ix1, ix2, q1, q02):
    mesh = plsc.VectorSubcoreMesh(core_axis_name="c", subcore_axis_name="s")
    f = pl.kernel(
        _sc_body,
        out_type=[
            jax.ShapeDtypeStruct((B,), jnp.float32),
            jax.ShapeDtypeStruct((NW, 3, L), jnp.float32),
        ],
        mesh=mesh,
        compiler_params=pltpu.CompilerParams(
            needs_layout_passes=False, use_tc_tiling_on_sc=True),
        scratch_types=[
            pltpu.VMEM((PER_W,), jnp.int32),
            pltpu.VMEM((PER_W,), jnp.int32),
            pltpu.VMEM((PER_W,), jnp.int32),
            pltpu.VMEM((C, R * R), jnp.float32),
            pltpu.VMEM((C, 8 * L), jnp.float32),
            pltpu.VMEM((C, 8 * L), jnp.float32),
            pltpu.VMEM((C, R * R), jnp.float32),
            pltpu.VMEM((C, 8 * L), jnp.float32),
            pltpu.VMEM((C, 8 * L), jnp.float32),
            pltpu.VMEM((C,), jnp.float32),
            pltpu.VMEM((3, L), jnp.float32),
            pltpu.SemaphoreType.DMA,
            pltpu.SemaphoreType.DMA,
        ],
    )
    return f(ix0, ix1, ix2, q1, q02)


def kernel(indices, W0, W1, W2, P0, P1, P2):
    ix0 = indices[:, 0]
    ix1 = indices[:, 1]
    ix2 = indices[:, 2]
    q1 = jnp.transpose(W1 * P1, (1, 0, 2)).reshape(N, R * R)
    q02 = jnp.concatenate(
        [W0[0] * P0[0],
         jnp.transpose(W2[:, :, 0] * P2[:, :, 0], (1, 0)),
         jnp.zeros((N, 6 * R), jnp.float32)], axis=1)
    preds, partials = _tt_lookup(ix0, ix1, ix2, q1, q02)
    s = jnp.sum(partials, axis=(0, 2))
    reg = REG_PARA * (jnp.abs(s[0]) / (B * R)
                      + jnp.abs(s[1]) / (B * R * R)
                      + jnp.abs(s[2]) / (B * R))
    return preds, reg
